# baseline (speedup 1.0000x reference)
import functools

import jax
import jax.numpy as jnp
from jax import lax
from jax.experimental import pallas as pl
from jax.experimental.pallas import tpu as pltpu


def _rvq_body(x_ref, cb_ref, hi_ref, mid_ref, lo_ref, csq_ref,
              tq_ref, codes_ref, loss_ref, *, ncb, k, tt, d, n_total):
    b_id = pl.program_id(0)
    t_id = pl.program_id(1)
    first = jnp.logical_and(b_id == 0, t_id == 0)
    last = jnp.logical_and(b_id == pl.num_programs(0) - 1,
                           t_id == pl.num_programs(1) - 1)

    @pl.when(first)
    def _():
        loss_ref[:, :] = jnp.zeros((1, 1), jnp.float32)

    r = x_ref[0].T
    xsq = jnp.sum(r * r, axis=1, keepdims=True)
    tq = jnp.zeros((tt, d), jnp.float32)
    iota = lax.broadcasted_iota(jnp.int32, (tt, k), 1)
    loss_part = jnp.float32(0.0)

    for i in range(ncb):
        cb = cb_ref[i]
        mm = lax.dot_general(r, cb, (((1,), (1,)), ((), ())),
                             preferred_element_type=jnp.float32)
        dist = (xsq - 2.0 * mm) + csq_ref[i][None, :]
        mn = jnp.min(dist, axis=1, keepdims=True)
        idx = jnp.min(jnp.where(dist == mn, iota, k), axis=1)
        codes_ref[0, i, :] = idx
        oh = (iota == idx[:, None]).astype(jnp.bfloat16)
        q = (lax.dot_general(oh, hi_ref[i], (((1,), (0,)), ((), ())),
                             preferred_element_type=jnp.float32)
             + lax.dot_general(oh, mid_ref[i], (((1,), (0,)), ((), ())),
                               preferred_element_type=jnp.float32)) \
            + lax.dot_general(oh, lo_ref[i], (((1,), (0,)), ((), ())),
                              preferred_element_type=jnp.float32)
        r = r - q
        tq = tq + q
        xsq = jnp.sum(r * r, axis=1, keepdims=True)
        loss_part = loss_part + jnp.sum(xsq)

    tq_ref[0] = tq.T
    loss_ref[:, :] = loss_ref[:, :] + loss_part.reshape(1, 1)

    @pl.when(last)
    def _():
        loss_ref[:, :] = loss_ref[:, :] * jnp.float32(1.0 / n_total)


def kernel(x, codebooks):
    b, d, t = x.shape
    ncb, k, _ = codebooks.shape
    tt = min(t, 512)
    assert t % tt == 0

    hi = codebooks.astype(jnp.bfloat16)
    r1 = codebooks - hi.astype(jnp.float32)
    mid = r1.astype(jnp.bfloat16)
    lo = (r1 - mid.astype(jnp.float32)).astype(jnp.bfloat16)
    csq = jnp.sum(codebooks ** 2, axis=-1)

    grid = (b, t // tt)
    body = functools.partial(_rvq_body, ncb=ncb, k=k, tt=tt, d=d,
                             n_total=b * d * t)
    tq, codes, loss = pl.pallas_call(
        body,
        grid=grid,
        in_specs=[
            pl.BlockSpec((1, d, tt), lambda bi, ti: (bi, 0, ti)),
            pl.BlockSpec((ncb, k, d), lambda bi, ti: (0, 0, 0)),
            pl.BlockSpec((ncb, k, d), lambda bi, ti: (0, 0, 0)),
            pl.BlockSpec((ncb, k, d), lambda bi, ti: (0, 0, 0)),
            pl.BlockSpec((ncb, k, d), lambda bi, ti: (0, 0, 0)),
            pl.BlockSpec((ncb, k), lambda bi, ti: (0, 0)),
        ],
        out_specs=[
            pl.BlockSpec((1, d, tt), lambda bi, ti: (bi, 0, ti)),
            pl.BlockSpec((1, ncb, tt), lambda bi, ti: (bi, 0, ti)),
            pl.BlockSpec((1, 1), lambda bi, ti: (0, 0)),
        ],
        out_shape=[
            jax.ShapeDtypeStruct((b, d, t), jnp.float32),
            jax.ShapeDtypeStruct((b, ncb, t), jnp.int32),
            jax.ShapeDtypeStruct((1, 1), jnp.float32),
        ],
    )(x, codebooks, hi, mid, lo, csq)
    return tq, codes, loss[0, 0]


# speedup vs baseline: 1.6961x; 1.6961x over previous
import functools

import jax
import jax.numpy as jnp
from jax import lax
from jax.experimental import pallas as pl
from jax.experimental.pallas import tpu as pltpu

_NS = 4


def _rvq_body(x_ref, cb_ref, hi_ref, mid_ref, lo_ref, csq_ref,
              tq_ref, codes_ref, loss_ref, *, ncb, k, tt, d, n_total):
    b_id = pl.program_id(0)
    t_id = pl.program_id(1)
    first = jnp.logical_and(b_id == 0, t_id == 0)
    last = jnp.logical_and(b_id == pl.num_programs(0) - 1,
                           t_id == pl.num_programs(1) - 1)

    @pl.when(first)
    def _():
        loss_ref[:, :] = jnp.zeros((1, 1), jnp.float32)

    ns = _NS
    st = tt // ns
    sub = range(ns)
    rs = [x_ref[0, :, s * st:(s + 1) * st].T for s in sub]
    xsqs = [jnp.sum(r * r, axis=1, keepdims=True) for r in rs]
    tqs = [jnp.zeros((st, d), jnp.float32) for _ in sub]
    iotaf = lax.broadcasted_iota(jnp.int32, (st, k), 1).astype(jnp.float32)
    loss_part = jnp.float32(0.0)

    for i in range(ncb):
        cb = cb_ref[i]
        mms = [lax.dot_general(rs[s], cb, (((1,), (1,)), ((), ())),
                               preferred_element_type=jnp.float32)
               for s in sub]
        dists = [(xsqs[s] - 2.0 * mms[s]) + csq_ref[i][None, :] for s in sub]
        mns = [jnp.min(dists[s], axis=1, keepdims=True) for s in sub]
        keys = [jnp.where(dists[s] == mns[s], iotaf, jnp.float32(2 * k))
                for s in sub]
        idxfs = [jnp.min(keys[s], axis=1, keepdims=True) for s in sub]
        for s in sub:
            codes_ref[0, i, s * st:(s + 1) * st] = idxfs[s][:, 0].astype(jnp.int32)
        ohs = [(keys[s] == idxfs[s]).astype(jnp.bfloat16) for s in sub]
        qs = [(lax.dot_general(ohs[s], hi_ref[i], (((1,), (0,)), ((), ())),
                               preferred_element_type=jnp.float32)
               + lax.dot_general(ohs[s], mid_ref[i], (((1,), (0,)), ((), ())),
                                 preferred_element_type=jnp.float32))
              + lax.dot_general(ohs[s], lo_ref[i], (((1,), (0,)), ((), ())),
                                preferred_element_type=jnp.float32)
              for s in sub]
        rs = [rs[s] - qs[s] for s in sub]
        tqs = [tqs[s] + qs[s] for s in sub]
        xsqs = [jnp.sum(r * r, axis=1, keepdims=True) for r in rs]
        loss_part = loss_part + sum(jnp.sum(xsq) for xsq in xsqs)

    for s in sub:
        tq_ref[0, :, s * st:(s + 1) * st] = tqs[s].T
    loss_ref[:, :] = loss_ref[:, :] + loss_part.reshape(1, 1)

    @pl.when(last)
    def _():
        loss_ref[:, :] = loss_ref[:, :] * jnp.float32(1.0 / n_total)


def kernel(x, codebooks):
    b, d, t = x.shape
    ncb, k, _ = codebooks.shape
    tt = min(t, 512)
    assert t % tt == 0

    hi = codebooks.astype(jnp.bfloat16)
    r1 = codebooks - hi.astype(jnp.float32)
    mid = r1.astype(jnp.bfloat16)
    lo = (r1 - mid.astype(jnp.float32)).astype(jnp.bfloat16)
    csq = jnp.sum(codebooks ** 2, axis=-1)

    grid = (b, t // tt)
    body = functools.partial(_rvq_body, ncb=ncb, k=k, tt=tt, d=d,
                             n_total=b * d * t)
    tq, codes, loss = pl.pallas_call(
        body,
        grid=grid,
        in_specs=[
            pl.BlockSpec((1, d, tt), lambda bi, ti: (bi, 0, ti)),
            pl.BlockSpec((ncb, k, d), lambda bi, ti: (0, 0, 0)),
            pl.BlockSpec((ncb, k, d), lambda bi, ti: (0, 0, 0)),
            pl.BlockSpec((ncb, k, d), lambda bi, ti: (0, 0, 0)),
            pl.BlockSpec((ncb, k, d), lambda bi, ti: (0, 0, 0)),
            pl.BlockSpec((ncb, k), lambda bi, ti: (0, 0)),
        ],
        out_specs=[
            pl.BlockSpec((1, d, tt), lambda bi, ti: (bi, 0, ti)),
            pl.BlockSpec((1, ncb, tt), lambda bi, ti: (bi, 0, ti)),
            pl.BlockSpec((1, 1), lambda bi, ti: (0, 0)),
        ],
        out_shape=[
            jax.ShapeDtypeStruct((b, d, t), jnp.float32),
            jax.ShapeDtypeStruct((b, ncb, t), jnp.int32),
            jax.ShapeDtypeStruct((1, 1), jnp.float32),
        ],
    )(x, codebooks, hi, mid, lo, csq)
    return tq, codes, loss[0, 0]
